# baseline (speedup 1.0000x reference)
import functools
import jax
import jax.numpy as jnp
from jax import lax
from jax.experimental import pallas as pl
from jax.experimental.pallas import tpu as pltpu
from jax.experimental.pallas import tpu_sc as plsc

_BLOCK = 10000
_CHUNK = 80


def _sc_gather(node_type, q_table):
    n = node_type.shape[0]
    n_types, d_out = q_table.shape
    n_chunks = n // _CHUNK
    info = plsc.get_sparse_core_info()
    nc, ns = info.num_cores, info.num_subcores
    nw = nc * ns
    iters = pl.cdiv(n_chunks, nw)
    mesh = plsc.VectorSubcoreMesh(core_axis_name="c", subcore_axis_name="s")

    @functools.partial(
        pl.kernel, mesh=mesh,
        out_type=jax.ShapeDtypeStruct((n, d_out), jnp.float32),
        scratch_types=[
            pltpu.VMEM((_CHUNK,), jnp.int32),
            pltpu.VMEM((_CHUNK, d_out), jnp.float32),
            pltpu.SemaphoreType.DMA,
        ],
    )
    def gather_kernel(nt_hbm, q_hbm, out_hbm, idx_v, rows_v, sem):
        wid = lax.axis_index("s") * nc + lax.axis_index("c")

        def body(t, _):
            c = wid + t * nw

            @pl.when(c < n_chunks)
            def _():
                base = pl.multiple_of(c * _CHUNK, 8)
                pltpu.sync_copy(nt_hbm.at[pl.ds(base, _CHUNK)], idx_v)
                pltpu.async_copy(q_hbm.at[idx_v], rows_v, sem).wait()
                pltpu.sync_copy(rows_v, out_hbm.at[pl.ds(base, _CHUNK)])

            return None

        lax.fori_loop(0, iters, body, None)

    return gather_kernel(node_type.astype(jnp.int32), q_table)


def _tc_kernel(x_ref, qg_ref, wt_ref, o_ref):
    o_ref[...] = qg_ref[...] + jax.lax.dot_general(
        x_ref[...], wt_ref[...], (((1,), (0,)), ((), ())),
        preferred_element_type=jnp.float32,
        precision=jax.lax.Precision.DEFAULT)


def kernel(x, node_type, q_table, b_weight):
    n, d_bits = x.shape
    n_types, d_out = q_table.shape
    qg = _sc_gather(node_type, q_table)
    bsz = _BLOCK
    nb = pl.cdiv(n, bsz)
    wt = b_weight.T
    return pl.pallas_call(
        _tc_kernel,
        grid=(nb,),
        in_specs=[
            pl.BlockSpec((bsz, d_bits), lambda i: (i, 0)),
            pl.BlockSpec((bsz, d_out), lambda i: (i, 0)),
            pl.BlockSpec((d_bits, d_out), lambda i: (0, 0)),
        ],
        out_specs=pl.BlockSpec((bsz, d_out), lambda i: (i, 0)),
        out_shape=jax.ShapeDtypeStruct((n, d_out), jnp.float32),
        compiler_params=pltpu.CompilerParams(
            dimension_semantics=("parallel",)),
    )(x, qg, wt)


# speedup vs baseline: 4.1061x; 4.1061x over previous
import jax
import jax.numpy as jnp
from jax.experimental import pallas as pl
from jax.experimental.pallas import tpu as pltpu

_BLOCK = 16384


def _fused_kernel(nt_ref, x_ref, wt_ref, q_ref, o_ref):
    xb = x_ref[...]
    nt = nt_ref[0]
    bsz = xb.shape[0]
    n_types = q_ref.shape[0]
    oh_t = (jax.lax.broadcasted_iota(jnp.int32, (n_types, bsz), 0) == nt
            ).astype(jnp.float32)
    acc = jax.lax.dot_general(
        xb, wt_ref[...], (((1,), (0,)), ((), ())),
        preferred_element_type=jnp.float32,
        precision=jax.lax.Precision.DEFAULT)
    acc = acc + jax.lax.dot_general(
        oh_t, q_ref[...], (((0,), (0,)), ((), ())),
        preferred_element_type=jnp.float32,
        precision=jax.lax.Precision.DEFAULT)
    o_ref[...] = acc


def kernel(x, node_type, q_table, b_weight):
    n, d_bits = x.shape
    n_types, d_out = q_table.shape
    bsz = _BLOCK
    nb = pl.cdiv(n, bsz)
    n_pad = nb * bsz
    nt3 = jnp.pad(node_type.astype(jnp.int32), (0, n_pad - n)).reshape(
        nb, 1, bsz)
    wt = b_weight.T
    return pl.pallas_call(
        _fused_kernel,
        grid=(nb,),
        in_specs=[
            pl.BlockSpec((1, 1, bsz), lambda i: (i, 0, 0)),
            pl.BlockSpec((bsz, d_bits), lambda i: (i, 0)),
            pl.BlockSpec((d_bits, d_out), lambda i: (0, 0)),
            pl.BlockSpec((n_types, d_out), lambda i: (0, 0)),
        ],
        out_specs=pl.BlockSpec((bsz, d_out), lambda i: (i, 0)),
        out_shape=jax.ShapeDtypeStruct((n, d_out), jnp.float32),
        compiler_params=pltpu.CompilerParams(
            dimension_semantics=("parallel",)),
    )(nt3, x, wt, q_table)
